# Initial kernel scaffold; baseline (speedup 1.0000x reference)
import jax
import jax.numpy as jnp
from jax.experimental import pallas as pl


def kernel(x, edge_index, W0, b0, W1, b1, W2, b2, W3, b3):
    raise NotImplementedError("write your pallas kernel here")


# speedup vs baseline: 12.0665x; 12.0665x over previous
import functools

import jax
import jax.numpy as jnp
from jax import lax
from jax.experimental import pallas as pl
from jax.experimental.pallas import tpu as pltpu
from jax.experimental.pallas import tpu_sc as plsc

N = 10000
E = 320000
D = 128

NC = 2
NS = 16
NW = NC * NS
EPW = E // NW
WIN = 128
NWIN = EPW // WIN
TAIL = EPW - NWIN * WIN
RPS = 624
RPS_LAST = N - 15 * RPS

_mesh = plsc.VectorSubcoreMesh(core_axis_name="c", subcore_axis_name="s")


def _rows_foreach_subcore(s, fn):
    off = pl.multiple_of(s * RPS, 8)

    @pl.when(s < NS - 1)
    def _():
        fn(off, RPS)

    @pl.when(s == NS - 1)
    def _():
        fn(off, RPS_LAST)


@functools.partial(
    pl.kernel,
    out_type=jax.ShapeDtypeStruct((NC, N, D), jnp.float32),
    mesh=_mesh,
    scratch_types=[
        pltpu.VMEM((WIN,), jnp.int32),
        pltpu.VMEM((TAIL,), jnp.int32),
        pltpu.VMEM((WIN, D), jnp.float32),
        pltpu.VMEM((TAIL, D), jnp.float32),
        pltpu.VMEM_SHARED((N, D), jnp.float32),
    ],
)
def _sc_degree(dst_hbm, zeros_hbm, ones_hbm, out_hbm,
               idx_v, idxt_v, ones_v, onest_v, acc_sh):
    c = lax.axis_index("c")
    s = lax.axis_index("s")
    w = c * NS + s
    pltpu.sync_copy(ones_hbm, ones_v)
    pltpu.sync_copy(ones_hbm.at[pl.ds(0, TAIL)], onest_v)
    _rows_foreach_subcore(s, lambda off, nr: pltpu.sync_copy(
        zeros_hbm.at[pl.ds(off, nr)], acc_sh.at[pl.ds(off, nr)]))
    plsc.subcore_barrier()
    base = w * EPW

    @pl.loop(0, NWIN)
    def _(i):
        pltpu.sync_copy(dst_hbm.at[pl.ds(base + i * WIN, WIN)], idx_v)
        pltpu.sync_copy(ones_v, acc_sh.at[idx_v], add=True)

    pltpu.sync_copy(dst_hbm.at[pl.ds(base + NWIN * WIN, TAIL)], idxt_v)
    pltpu.sync_copy(onest_v, acc_sh.at[idxt_v], add=True)
    plsc.subcore_barrier()
    _rows_foreach_subcore(s, lambda off, nr: pltpu.sync_copy(
        acc_sh.at[pl.ds(off, nr)], out_hbm.at[c, pl.ds(off, nr)]))


@functools.partial(
    pl.kernel,
    out_type=jax.ShapeDtypeStruct((NC, N, D), jnp.float32),
    mesh=_mesh,
    scratch_types=[
        pltpu.VMEM((WIN,), jnp.int32),
        pltpu.VMEM((WIN,), jnp.int32),
        pltpu.VMEM((TAIL,), jnp.int32),
        pltpu.VMEM((TAIL,), jnp.int32),
        pltpu.VMEM((WIN, D), jnp.float32),
        pltpu.VMEM((TAIL, D), jnp.float32),
        pltpu.VMEM_SHARED((N, D), jnp.float32),
        pltpu.SemaphoreType.DMA,
    ],
)
def _sc_scatter(src_hbm, dst_hbm, g_hbm, zeros_hbm, out_hbm,
                src_v, dst_v, srct_v, dstt_v, rows_v, rowst_v, acc_sh, sem):
    c = lax.axis_index("c")
    s = lax.axis_index("s")
    w = c * NS + s
    _rows_foreach_subcore(s, lambda off, nr: pltpu.sync_copy(
        zeros_hbm.at[pl.ds(off, nr)], acc_sh.at[pl.ds(off, nr)]))
    plsc.subcore_barrier()
    base = w * EPW

    @pl.loop(0, NWIN)
    def _(i):
        pltpu.sync_copy(src_hbm.at[pl.ds(base + i * WIN, WIN)], src_v)
        pltpu.sync_copy(dst_hbm.at[pl.ds(base + i * WIN, WIN)], dst_v)
        pltpu.async_copy(g_hbm.at[src_v], rows_v, sem).wait()
        pltpu.sync_copy(rows_v, acc_sh.at[dst_v], add=True)

    pltpu.sync_copy(src_hbm.at[pl.ds(base + NWIN * WIN, TAIL)], srct_v)
    pltpu.sync_copy(dst_hbm.at[pl.ds(base + NWIN * WIN, TAIL)], dstt_v)
    pltpu.async_copy(g_hbm.at[srct_v], rowst_v, sem).wait()
    pltpu.sync_copy(rowst_v, acc_sh.at[dstt_v], add=True)
    plsc.subcore_barrier()
    _rows_foreach_subcore(s, lambda off, nr: pltpu.sync_copy(
        acc_sh.at[pl.ds(off, nr)], out_hbm.at[c, pl.ds(off, nr)]))


BLK = 2000


def _tc_first_body(degp_ref, x_ref, w_ref, g_ref, dinv_ref):
    deg = degp_ref[0][:, 0:1] + degp_ref[1][:, 0:1] + 1.0
    dinv = lax.rsqrt(deg)
    t = jnp.dot(x_ref[...], w_ref[...], preferred_element_type=jnp.float32,
                precision=lax.Precision.HIGHEST)
    g_ref[...] = t * dinv
    dinv_ref[...] = dinv


def _tc_mid_body(accp_ref, g_ref, dinv_ref, b_ref, w_ref, gnext_ref):
    h = (accp_ref[0] + accp_ref[1] + g_ref[...]) * dinv_ref[...] + b_ref[...]
    h = jnp.maximum(h, 0.0)
    gnext_ref[...] = jnp.dot(h, w_ref[...], preferred_element_type=jnp.float32,
                             precision=lax.Precision.HIGHEST) * dinv_ref[...]


def _tc_last_body(accp_ref, g_ref, dinv_ref, b_ref, out_ref):
    out_ref[...] = (accp_ref[0] + accp_ref[1] + g_ref[...]) * dinv_ref[...] \
        + b_ref[...]


def _tc_first(degp, x, w):
    return pl.pallas_call(
        _tc_first_body,
        grid=(N // BLK,),
        in_specs=[
            pl.BlockSpec((2, BLK, D), lambda i: (0, i, 0)),
            pl.BlockSpec((BLK, D), lambda i: (i, 0)),
            pl.BlockSpec((D, D), lambda i: (0, 0)),
        ],
        out_specs=[
            pl.BlockSpec((BLK, D), lambda i: (i, 0)),
            pl.BlockSpec((BLK, 1), lambda i: (i, 0)),
        ],
        out_shape=[
            jax.ShapeDtypeStruct((N, D), jnp.float32),
            jax.ShapeDtypeStruct((N, 1), jnp.float32),
        ],
    )(degp, x, w)


def _tc_mid(accp, g, dinv, b, w):
    return pl.pallas_call(
        _tc_mid_body,
        grid=(N // BLK,),
        in_specs=[
            pl.BlockSpec((2, BLK, D), lambda i: (0, i, 0)),
            pl.BlockSpec((BLK, D), lambda i: (i, 0)),
            pl.BlockSpec((BLK, 1), lambda i: (i, 0)),
            pl.BlockSpec((1, D), lambda i: (0, 0)),
            pl.BlockSpec((D, D), lambda i: (0, 0)),
        ],
        out_specs=pl.BlockSpec((BLK, D), lambda i: (i, 0)),
        out_shape=jax.ShapeDtypeStruct((N, D), jnp.float32),
    )(accp, g, dinv, b, w)


def _tc_last(accp, g, dinv, b):
    return pl.pallas_call(
        _tc_last_body,
        grid=(N // BLK,),
        in_specs=[
            pl.BlockSpec((2, BLK, D), lambda i: (0, i, 0)),
            pl.BlockSpec((BLK, D), lambda i: (i, 0)),
            pl.BlockSpec((BLK, 1), lambda i: (i, 0)),
            pl.BlockSpec((1, D), lambda i: (0, 0)),
        ],
        out_specs=pl.BlockSpec((BLK, D), lambda i: (i, 0)),
        out_shape=jax.ShapeDtypeStruct((N, D), jnp.float32),
    )(accp, g, dinv, b)


def kernel(x, edge_index, W0, b0, W1, b1, W2, b2, W3, b3):
    src = edge_index[0]
    dst = edge_index[1]
    zerosD = jnp.zeros((N, D), jnp.float32)
    onesW = jnp.ones((WIN, D), jnp.float32)

    degp = _sc_degree(dst, zerosD, onesW)
    g, dinv = _tc_first(degp, x, W0)
    for w_next, b_prev in ((W1, b0), (W2, b1), (W3, b2)):
        accp = _sc_scatter(src, dst, g, zerosD)
        g = _tc_mid(accp, g, dinv, b_prev.reshape(1, D), w_next)
    accp = _sc_scatter(src, dst, g, zerosD)
    return _tc_last(accp, g, dinv, b3.reshape(1, D))


# speedup vs baseline: 20.7438x; 1.7191x over previous
import functools

import jax
import jax.numpy as jnp
from jax import lax
from jax.experimental import pallas as pl
from jax.experimental.pallas import tpu as pltpu
from jax.experimental.pallas import tpu_sc as plsc

N = 10000
E = 320000
D = 128

NC = 2
NS = 16
NW = NC * NS
EPW = E // NW
WIN = 128
NWIN = EPW // WIN
TAIL = EPW - NWIN * WIN
RPS = 624
RPS_LAST = N - 15 * RPS

_mesh = plsc.VectorSubcoreMesh(core_axis_name="c", subcore_axis_name="s")


def _rows_foreach_subcore(s, fn):
    off = pl.multiple_of(s * RPS, 8)

    @pl.when(s < NS - 1)
    def _():
        fn(off, RPS)

    @pl.when(s == NS - 1)
    def _():
        fn(off, RPS_LAST)


@functools.partial(
    pl.kernel,
    out_type=jax.ShapeDtypeStruct((NC, N, D), jnp.float32),
    mesh=_mesh,
    scratch_types=[
        pltpu.VMEM((WIN,), jnp.int32),
        pltpu.VMEM((TAIL,), jnp.int32),
        pltpu.VMEM((WIN, D), jnp.float32),
        pltpu.VMEM((TAIL, D), jnp.float32),
        pltpu.VMEM_SHARED((N, D), jnp.float32),
    ],
)
def _sc_degree(dst_hbm, zeros_hbm, ones_hbm, out_hbm,
               idx_v, idxt_v, ones_v, onest_v, acc_sh):
    c = lax.axis_index("c")
    s = lax.axis_index("s")
    w = c * NS + s
    pltpu.sync_copy(ones_hbm, ones_v)
    pltpu.sync_copy(ones_hbm.at[pl.ds(0, TAIL)], onest_v)
    _rows_foreach_subcore(s, lambda off, nr: pltpu.sync_copy(
        zeros_hbm.at[pl.ds(off, nr)], acc_sh.at[pl.ds(off, nr)]))
    plsc.subcore_barrier()
    base = w * EPW

    @pl.loop(0, NWIN)
    def _(i):
        pltpu.sync_copy(dst_hbm.at[pl.ds(base + i * WIN, WIN)], idx_v)
        pltpu.sync_copy(ones_v, acc_sh.at[idx_v], add=True)

    pltpu.sync_copy(dst_hbm.at[pl.ds(base + NWIN * WIN, TAIL)], idxt_v)
    pltpu.sync_copy(onest_v, acc_sh.at[idxt_v], add=True)
    plsc.subcore_barrier()
    _rows_foreach_subcore(s, lambda off, nr: pltpu.sync_copy(
        acc_sh.at[pl.ds(off, nr)], out_hbm.at[c, pl.ds(off, nr)]))


@functools.partial(
    pl.kernel,
    out_type=jax.ShapeDtypeStruct((NC, N, D), jnp.float32),
    mesh=_mesh,
    scratch_types=[
        pltpu.VMEM((WIN,), jnp.int32),
        pltpu.VMEM((WIN,), jnp.int32),
        pltpu.VMEM((WIN,), jnp.int32),
        pltpu.VMEM((WIN,), jnp.int32),
        pltpu.VMEM((TAIL,), jnp.int32),
        pltpu.VMEM((TAIL,), jnp.int32),
        pltpu.VMEM((WIN, D), jnp.float32),
        pltpu.VMEM((WIN, D), jnp.float32),
        pltpu.VMEM((TAIL, D), jnp.float32),
        pltpu.VMEM_SHARED((N, D), jnp.float32),
        pltpu.SemaphoreType.DMA,
        pltpu.SemaphoreType.DMA,
        pltpu.SemaphoreType.DMA,
        pltpu.SemaphoreType.DMA,
        pltpu.SemaphoreType.DMA,
    ],
)
def _sc_scatter(src_hbm, dst_hbm, g_hbm, zeros_hbm, out_hbm,
                src0, dst0, src1, dst1, srct_v, dstt_v,
                rows0, rows1, rowst_v, acc_sh,
                si0, si1, sg0, sg1, semt):
    c = lax.axis_index("c")
    s = lax.axis_index("s")
    w = c * NS + s
    _rows_foreach_subcore(s, lambda off, nr: pltpu.sync_copy(
        zeros_hbm.at[pl.ds(off, nr)], acc_sh.at[pl.ds(off, nr)]))
    plsc.subcore_barrier()
    base = w * EPW

    def issue_idx(win, sv, dv, sem):
        pltpu.async_copy(src_hbm.at[pl.ds(base + win * WIN, WIN)], sv, sem)
        pltpu.async_copy(dst_hbm.at[pl.ds(base + win * WIN, WIN)], dv, sem)

    def wait_idx(sv, dv, sem):
        pltpu.make_async_copy(src_hbm.at[pl.ds(base, WIN)], sv, sem).wait()
        pltpu.make_async_copy(dst_hbm.at[pl.ds(base, WIN)], dv, sem).wait()

    def issue_gather(sv, rows, sem):
        pltpu.async_copy(g_hbm.at[sv], rows, sem)

    def wait_gather(sv, rows, sem):
        pltpu.make_async_copy(g_hbm.at[sv], rows, sem).wait()

    issue_idx(0, src0, dst0, si0)
    issue_idx(1, src1, dst1, si1)
    wait_idx(src0, dst0, si0)
    issue_gather(src0, rows0, sg0)
    wait_idx(src1, dst1, si1)
    issue_gather(src1, rows1, sg1)

    @pl.loop(0, (NWIN - 2) // 2)
    def _(k):
        wait_gather(src0, rows0, sg0)
        pltpu.sync_copy(rows0, acc_sh.at[dst0], add=True)
        issue_idx(2 * k + 2, src0, dst0, si0)
        wait_idx(src0, dst0, si0)
        issue_gather(src0, rows0, sg0)
        wait_gather(src1, rows1, sg1)
        pltpu.sync_copy(rows1, acc_sh.at[dst1], add=True)
        issue_idx(2 * k + 3, src1, dst1, si1)
        wait_idx(src1, dst1, si1)
        issue_gather(src1, rows1, sg1)

    pltpu.async_copy(src_hbm.at[pl.ds(base + NWIN * WIN, TAIL)], srct_v, semt)
    pltpu.async_copy(dst_hbm.at[pl.ds(base + NWIN * WIN, TAIL)], dstt_v, semt)
    wait_gather(src0, rows0, sg0)
    pltpu.sync_copy(rows0, acc_sh.at[dst0], add=True)
    pltpu.make_async_copy(src_hbm.at[pl.ds(base, TAIL)], srct_v, semt).wait()
    pltpu.make_async_copy(dst_hbm.at[pl.ds(base, TAIL)], dstt_v, semt).wait()
    issue_gather(srct_v, rowst_v, semt)
    wait_gather(src1, rows1, sg1)
    pltpu.sync_copy(rows1, acc_sh.at[dst1], add=True)
    pltpu.make_async_copy(g_hbm.at[srct_v], rowst_v, semt).wait()
    pltpu.sync_copy(rowst_v, acc_sh.at[dstt_v], add=True)
    plsc.subcore_barrier()
    _rows_foreach_subcore(s, lambda off, nr: pltpu.sync_copy(
        acc_sh.at[pl.ds(off, nr)], out_hbm.at[c, pl.ds(off, nr)]))


BLK = 2000


def _tc_first_body(degp_ref, x_ref, w_ref, g_ref, dinv_ref):
    deg = degp_ref[0][:, 0:1] + degp_ref[1][:, 0:1] + 1.0
    dinv = lax.rsqrt(deg)
    t = jnp.dot(x_ref[...], w_ref[...], preferred_element_type=jnp.float32,
                precision=lax.Precision.HIGHEST)
    g_ref[...] = t * dinv
    dinv_ref[...] = dinv


def _tc_mid_body(accp_ref, g_ref, dinv_ref, b_ref, w_ref, gnext_ref):
    h = (accp_ref[0] + accp_ref[1] + g_ref[...]) * dinv_ref[...] + b_ref[...]
    h = jnp.maximum(h, 0.0)
    gnext_ref[...] = jnp.dot(h, w_ref[...], preferred_element_type=jnp.float32,
                             precision=lax.Precision.HIGHEST) * dinv_ref[...]


def _tc_last_body(accp_ref, g_ref, dinv_ref, b_ref, out_ref):
    out_ref[...] = (accp_ref[0] + accp_ref[1] + g_ref[...]) * dinv_ref[...] \
        + b_ref[...]


def _tc_first(degp, x, w):
    return pl.pallas_call(
        _tc_first_body,
        grid=(N // BLK,),
        in_specs=[
            pl.BlockSpec((2, BLK, D), lambda i: (0, i, 0)),
            pl.BlockSpec((BLK, D), lambda i: (i, 0)),
            pl.BlockSpec((D, D), lambda i: (0, 0)),
        ],
        out_specs=[
            pl.BlockSpec((BLK, D), lambda i: (i, 0)),
            pl.BlockSpec((BLK, 1), lambda i: (i, 0)),
        ],
        out_shape=[
            jax.ShapeDtypeStruct((N, D), jnp.float32),
            jax.ShapeDtypeStruct((N, 1), jnp.float32),
        ],
    )(degp, x, w)


def _tc_mid(accp, g, dinv, b, w):
    return pl.pallas_call(
        _tc_mid_body,
        grid=(N // BLK,),
        in_specs=[
            pl.BlockSpec((2, BLK, D), lambda i: (0, i, 0)),
            pl.BlockSpec((BLK, D), lambda i: (i, 0)),
            pl.BlockSpec((BLK, 1), lambda i: (i, 0)),
            pl.BlockSpec((1, D), lambda i: (0, 0)),
            pl.BlockSpec((D, D), lambda i: (0, 0)),
        ],
        out_specs=pl.BlockSpec((BLK, D), lambda i: (i, 0)),
        out_shape=jax.ShapeDtypeStruct((N, D), jnp.float32),
    )(accp, g, dinv, b, w)


def _tc_last(accp, g, dinv, b):
    return pl.pallas_call(
        _tc_last_body,
        grid=(N // BLK,),
        in_specs=[
            pl.BlockSpec((2, BLK, D), lambda i: (0, i, 0)),
            pl.BlockSpec((BLK, D), lambda i: (i, 0)),
            pl.BlockSpec((BLK, 1), lambda i: (i, 0)),
            pl.BlockSpec((1, D), lambda i: (0, 0)),
        ],
        out_specs=pl.BlockSpec((BLK, D), lambda i: (i, 0)),
        out_shape=jax.ShapeDtypeStruct((N, D), jnp.float32),
    )(accp, g, dinv, b)


def kernel(x, edge_index, W0, b0, W1, b1, W2, b2, W3, b3):
    src = edge_index[0]
    dst = edge_index[1]
    zerosD = jnp.zeros((N, D), jnp.float32)
    onesW = jnp.ones((WIN, D), jnp.float32)

    degp = _sc_degree(dst, zerosD, onesW)
    g, dinv = _tc_first(degp, x, W0)
    for w_next, b_prev in ((W1, b0), (W2, b1), (W3, b2)):
        accp = _sc_scatter(src, dst, g, zerosD)
        g = _tc_mid(accp, g, dinv, b_prev.reshape(1, D), w_next)
    accp = _sc_scatter(src, dst, g, zerosD)
    return _tc_last(accp, g, dinv, b3.reshape(1, D))


# speedup vs baseline: 23.0883x; 1.1130x over previous
import functools

import jax
import jax.numpy as jnp
from jax import lax
from jax.experimental import pallas as pl
from jax.experimental.pallas import tpu as pltpu
from jax.experimental.pallas import tpu_sc as plsc

N = 10000
E = 320000
D = 128

NC = 2
NS = 16
NW = NC * NS
EPW = E // NW
WIN = 128
NWIN = EPW // WIN
TAIL = EPW - NWIN * WIN
RPS = 624
RPS_LAST = N - 15 * RPS

_mesh = plsc.VectorSubcoreMesh(core_axis_name="c", subcore_axis_name="s")


def _rows_foreach_subcore(s, fn):
    off = pl.multiple_of(s * RPS, 8)

    @pl.when(s < NS - 1)
    def _():
        fn(off, RPS)

    @pl.when(s == NS - 1)
    def _():
        fn(off, RPS_LAST)


@functools.partial(
    pl.kernel,
    out_type=jax.ShapeDtypeStruct((NC, N, D), jnp.float32),
    mesh=_mesh,
    scratch_types=[
        pltpu.VMEM((WIN,), jnp.int32),
        pltpu.VMEM((TAIL,), jnp.int32),
        pltpu.VMEM((WIN, D), jnp.float32),
        pltpu.VMEM((TAIL, D), jnp.float32),
        pltpu.VMEM_SHARED((N, D), jnp.float32),
    ],
)
def _sc_degree(dst_hbm, zeros_hbm, ones_hbm, out_hbm,
               idx_v, idxt_v, ones_v, onest_v, acc_sh):
    c = lax.axis_index("c")
    s = lax.axis_index("s")
    w = c * NS + s
    pltpu.sync_copy(ones_hbm, ones_v)
    pltpu.sync_copy(ones_hbm.at[pl.ds(0, TAIL)], onest_v)
    _rows_foreach_subcore(s, lambda off, nr: pltpu.sync_copy(
        zeros_hbm.at[pl.ds(off, nr)], acc_sh.at[pl.ds(off, nr)]))
    plsc.subcore_barrier()
    base = w * EPW

    @pl.loop(0, NWIN)
    def _(i):
        pltpu.sync_copy(dst_hbm.at[pl.ds(base + i * WIN, WIN)], idx_v)
        pltpu.sync_copy(ones_v, acc_sh.at[idx_v], add=True)

    pltpu.sync_copy(dst_hbm.at[pl.ds(base + NWIN * WIN, TAIL)], idxt_v)
    pltpu.sync_copy(onest_v, acc_sh.at[idxt_v], add=True)
    plsc.subcore_barrier()
    _rows_foreach_subcore(s, lambda off, nr: pltpu.sync_copy(
        acc_sh.at[pl.ds(off, nr)], out_hbm.at[c, pl.ds(off, nr)]))


NR = 3
NI = 8
SB = 24
PEEL = SB
NSTEADY = (NWIN - PEEL - 6) // SB
EPI = NWIN - PEEL - NSTEADY * SB


@functools.partial(
    pl.kernel,
    out_type=jax.ShapeDtypeStruct((NC, N, D), jnp.float32),
    mesh=_mesh,
    scratch_types=(
        [pltpu.VMEM((WIN,), jnp.int32)] * NR
        + [pltpu.VMEM((WIN,), jnp.int32)] * NI
        + [pltpu.VMEM((WIN, D), jnp.float32)] * NR
        + [pltpu.VMEM((TAIL,), jnp.int32)] * 2
        + [pltpu.VMEM_SHARED((N, D), jnp.float32)]
        + [pltpu.SemaphoreType.DMA] * (NR + NR + NI + 1)
    ),
)
def _sc_scatter(src_hbm, dst_hbm, g_hbm, zeros_hbm, out_hbm, *scr):
    srcb = scr[0:NR]
    dstb = scr[NR:NR + NI]
    rows = scr[NR + NI:2 * NR + NI]
    srct_v, dstt_v = scr[2 * NR + NI:2 * NR + NI + 2]
    acc_sh = scr[2 * NR + NI + 2]
    sem0 = 2 * NR + NI + 3
    sg = scr[sem0:sem0 + NR]
    ss = scr[sem0 + NR:sem0 + 2 * NR]
    si = scr[sem0 + 2 * NR:sem0 + 2 * NR + NI]
    semt = scr[sem0 + 2 * NR + NI]

    c = lax.axis_index("c")
    s = lax.axis_index("s")
    wkr = c * NS + s
    _rows_foreach_subcore(s, lambda off, nr: pltpu.sync_copy(
        zeros_hbm.at[pl.ds(off, nr)], acc_sh.at[pl.ds(off, nr)]))
    plsc.subcore_barrier()
    base = wkr * EPW

    def issue_idx(w, sw):
        pltpu.async_copy(src_hbm.at[pl.ds(base + w * WIN, WIN)],
                         srcb[sw % NR], si[sw % NI])
        pltpu.async_copy(dst_hbm.at[pl.ds(base + w * WIN, WIN)],
                         dstb[sw % NI], si[sw % NI])

    def wait_idx(sw):
        pltpu.make_async_copy(src_hbm.at[pl.ds(base, WIN)],
                              srcb[sw % NR], si[sw % NI]).wait()
        pltpu.make_async_copy(dst_hbm.at[pl.ds(base, WIN)],
                              dstb[sw % NI], si[sw % NI]).wait()

    def issue_gather(sw):
        pltpu.async_copy(g_hbm.at[srcb[sw % NR]], rows[sw % NR], sg[sw % NR])

    def wait_gather(sw):
        pltpu.make_async_copy(g_hbm.at[srcb[sw % NR]], rows[sw % NR],
                              sg[sw % NR]).wait()

    def issue_scatter(sw):
        pltpu.async_copy(rows[sw % NR], acc_sh.at[dstb[sw % NI]],
                         ss[sw % NR], add=True)

    def wait_scatter(sw):
        pltpu.make_async_copy(rows[sw % NR], acc_sh.at[dstb[sw % NI]],
                              ss[sw % NR]).wait()

    def body(w, sw, later_idx, later_gather, first):
        wait_gather(sw)
        issue_scatter(sw)
        if later_idx:
            issue_idx(w + NR, sw + NR)
        if not first:
            wait_scatter(sw - 1)
        if later_gather:
            wait_idx(sw + 2)
            issue_gather(sw + 2)

    for w in range(NR):
        issue_idx(w, w)
    for w in range(NR):
        wait_idx(w)
        issue_gather(w)

    for j in range(PEEL):
        body(j, j, later_idx=True, later_gather=(j >= 1), first=(j == 0))

    @pl.loop(0, NSTEADY)
    def _(k):
        w0 = PEEL + k * SB
        for j in range(SB):
            body(w0 + j, j, later_idx=True, later_gather=True, first=False)

    e0 = PEEL + NSTEADY * SB
    for j in range(e0, NWIN):
        body(j, j, later_idx=(j + NR < NWIN), later_gather=(j + 2 < NWIN),
             first=False)

    pltpu.async_copy(src_hbm.at[pl.ds(base + NWIN * WIN, TAIL)], srct_v, semt)
    pltpu.async_copy(dst_hbm.at[pl.ds(base + NWIN * WIN, TAIL)], dstt_v, semt)
    wait_scatter(NWIN - 1)
    pltpu.make_async_copy(src_hbm.at[pl.ds(base, TAIL)], srct_v, semt).wait()
    pltpu.make_async_copy(dst_hbm.at[pl.ds(base, TAIL)], dstt_v, semt).wait()
    tail_rows = rows[0].at[pl.ds(0, TAIL)]
    pltpu.async_copy(g_hbm.at[srct_v], tail_rows, semt)
    pltpu.make_async_copy(g_hbm.at[srct_v], tail_rows, semt).wait()
    pltpu.sync_copy(tail_rows, acc_sh.at[dstt_v], add=True)
    plsc.subcore_barrier()
    _rows_foreach_subcore(s, lambda off, nr: pltpu.sync_copy(
        acc_sh.at[pl.ds(off, nr)], out_hbm.at[c, pl.ds(off, nr)]))


BLK = 2000


def _tc_first_body(degp_ref, x_ref, w_ref, g_ref, dinv_ref):
    deg = degp_ref[0][:, 0:1] + degp_ref[1][:, 0:1] + 1.0
    dinv = lax.rsqrt(deg)
    t = jnp.dot(x_ref[...], w_ref[...], preferred_element_type=jnp.float32,
                precision=lax.Precision.HIGHEST)
    g_ref[...] = t * dinv
    dinv_ref[...] = dinv


def _tc_mid_body(accp_ref, g_ref, dinv_ref, b_ref, w_ref, gnext_ref):
    h = (accp_ref[0] + accp_ref[1] + g_ref[...]) * dinv_ref[...] + b_ref[...]
    h = jnp.maximum(h, 0.0)
    gnext_ref[...] = jnp.dot(h, w_ref[...], preferred_element_type=jnp.float32,
                             precision=lax.Precision.HIGHEST) * dinv_ref[...]


def _tc_last_body(accp_ref, g_ref, dinv_ref, b_ref, out_ref):
    out_ref[...] = (accp_ref[0] + accp_ref[1] + g_ref[...]) * dinv_ref[...] \
        + b_ref[...]


def _tc_first(degp, x, w):
    return pl.pallas_call(
        _tc_first_body,
        grid=(N // BLK,),
        in_specs=[
            pl.BlockSpec((2, BLK, D), lambda i: (0, i, 0)),
            pl.BlockSpec((BLK, D), lambda i: (i, 0)),
            pl.BlockSpec((D, D), lambda i: (0, 0)),
        ],
        out_specs=[
            pl.BlockSpec((BLK, D), lambda i: (i, 0)),
            pl.BlockSpec((BLK, 1), lambda i: (i, 0)),
        ],
        out_shape=[
            jax.ShapeDtypeStruct((N, D), jnp.float32),
            jax.ShapeDtypeStruct((N, 1), jnp.float32),
        ],
    )(degp, x, w)


def _tc_mid(accp, g, dinv, b, w):
    return pl.pallas_call(
        _tc_mid_body,
        grid=(N // BLK,),
        in_specs=[
            pl.BlockSpec((2, BLK, D), lambda i: (0, i, 0)),
            pl.BlockSpec((BLK, D), lambda i: (i, 0)),
            pl.BlockSpec((BLK, 1), lambda i: (i, 0)),
            pl.BlockSpec((1, D), lambda i: (0, 0)),
            pl.BlockSpec((D, D), lambda i: (0, 0)),
        ],
        out_specs=pl.BlockSpec((BLK, D), lambda i: (i, 0)),
        out_shape=jax.ShapeDtypeStruct((N, D), jnp.float32),
    )(accp, g, dinv, b, w)


def _tc_last(accp, g, dinv, b):
    return pl.pallas_call(
        _tc_last_body,
        grid=(N // BLK,),
        in_specs=[
            pl.BlockSpec((2, BLK, D), lambda i: (0, i, 0)),
            pl.BlockSpec((BLK, D), lambda i: (i, 0)),
            pl.BlockSpec((BLK, 1), lambda i: (i, 0)),
            pl.BlockSpec((1, D), lambda i: (0, 0)),
        ],
        out_specs=pl.BlockSpec((BLK, D), lambda i: (i, 0)),
        out_shape=jax.ShapeDtypeStruct((N, D), jnp.float32),
    )(accp, g, dinv, b)


def kernel(x, edge_index, W0, b0, W1, b1, W2, b2, W3, b3):
    src = edge_index[0]
    dst = edge_index[1]
    zerosD = jnp.zeros((N, D), jnp.float32)
    onesW = jnp.ones((WIN, D), jnp.float32)

    degp = _sc_degree(dst, zerosD, onesW)
    g, dinv = _tc_first(degp, x, W0)
    for w_next, b_prev in ((W1, b0), (W2, b1), (W3, b2)):
        accp = _sc_scatter(src, dst, g, zerosD)
        g = _tc_mid(accp, g, dinv, b_prev.reshape(1, D), w_next)
    accp = _sc_scatter(src, dst, g, zerosD)
    return _tc_last(accp, g, dinv, b3.reshape(1, D))


# speedup vs baseline: 24.4538x; 1.0591x over previous
import functools

import jax
import jax.numpy as jnp
from jax import lax
from jax.experimental import pallas as pl
from jax.experimental.pallas import tpu as pltpu
from jax.experimental.pallas import tpu_sc as plsc

N = 10000
E = 320000
D = 128

NC = 2
NS = 16
NW = NC * NS
EPW = E // NW
WIN = 128
NWIN = EPW // WIN
TAIL = EPW - NWIN * WIN
RPS = 624
RPS_LAST = N - 15 * RPS

_mesh = plsc.VectorSubcoreMesh(core_axis_name="c", subcore_axis_name="s")


def _rows_foreach_subcore(s, fn):
    off = pl.multiple_of(s * RPS, 8)

    @pl.when(s < NS - 1)
    def _():
        fn(off, RPS)

    @pl.when(s == NS - 1)
    def _():
        fn(off, RPS_LAST)


DNI = 8
DSB = 8


@functools.partial(
    pl.kernel,
    out_type=jax.ShapeDtypeStruct((NC, N, D), jnp.float32),
    mesh=_mesh,
    scratch_types=(
        [pltpu.VMEM((WIN,), jnp.int32)] * DNI
        + [pltpu.VMEM((TAIL,), jnp.int32)]
        + [pltpu.VMEM((WIN, D), jnp.float32)]
        + [pltpu.VMEM((TAIL, D), jnp.float32)]
        + [pltpu.VMEM_SHARED((N, D), jnp.float32)]
        + [pltpu.SemaphoreType.DMA] * (2 * DNI + 1)
    ),
)
def _sc_degree(dst_hbm, zeros_hbm, ones_hbm, out_hbm, *scr):
    dstb = scr[0:DNI]
    idxt_v = scr[DNI]
    ones_v = scr[DNI + 1]
    onest_v = scr[DNI + 2]
    acc_sh = scr[DNI + 3]
    si = scr[DNI + 4:2 * DNI + 4]
    ss = scr[2 * DNI + 4:3 * DNI + 4]
    semt = scr[3 * DNI + 4]

    c = lax.axis_index("c")
    s = lax.axis_index("s")
    wkr = c * NS + s
    pltpu.sync_copy(ones_hbm, ones_v)
    pltpu.sync_copy(ones_hbm.at[pl.ds(0, TAIL)], onest_v)
    _rows_foreach_subcore(s, lambda off, nr: pltpu.sync_copy(
        zeros_hbm.at[pl.ds(off, nr)], acc_sh.at[pl.ds(off, nr)]))
    plsc.subcore_barrier()
    base = wkr * EPW

    def issue_idx(w, sw):
        pltpu.async_copy(dst_hbm.at[pl.ds(base + w * WIN, WIN)],
                         dstb[sw % DNI], si[sw % DNI])

    def wait_idx(sw):
        pltpu.make_async_copy(dst_hbm.at[pl.ds(base, WIN)],
                              dstb[sw % DNI], si[sw % DNI]).wait()

    def issue_scatter(sw):
        pltpu.async_copy(ones_v, acc_sh.at[dstb[sw % DNI]], ss[sw % DNI],
                         add=True)

    def wait_scatter(sw):
        pltpu.make_async_copy(ones_v, acc_sh.at[dstb[sw % DNI]],
                              ss[sw % DNI]).wait()

    def body(w, sw, later_idx, first):
        wait_idx(sw)
        issue_scatter(sw)
        if later_idx:
            if not first:
                wait_scatter(sw + 4)
            issue_idx(w + 4, sw + 4)

    for w in range(4):
        issue_idx(w, w)
    for j in range(DSB):
        body(j, j, later_idx=True, first=(j < 4))

    @pl.loop(0, (NWIN - DSB - 6) // DSB)
    def _(k):
        w0 = DSB + k * DSB
        for j in range(DSB):
            body(w0 + j, j, later_idx=True, first=False)

    e0 = DSB + ((NWIN - DSB - 6) // DSB) * DSB
    for j in range(e0, NWIN):
        body(j, j, later_idx=(j + 4 < NWIN), first=False)

    pltpu.async_copy(dst_hbm.at[pl.ds(base + NWIN * WIN, TAIL)], idxt_v, semt)
    pltpu.make_async_copy(dst_hbm.at[pl.ds(base, TAIL)], idxt_v, semt).wait()
    pltpu.sync_copy(onest_v, acc_sh.at[idxt_v], add=True)
    for w in range(NWIN - 8, NWIN):
        wait_scatter(w % DSB)
    plsc.subcore_barrier()
    _rows_foreach_subcore(s, lambda off, nr: pltpu.sync_copy(
        acc_sh.at[pl.ds(off, nr)], out_hbm.at[c, pl.ds(off, nr)]))


NR = 3
NI = 8
SB = 24
PEEL = SB
NSTEADY = (NWIN - PEEL - 6) // SB
EPI = NWIN - PEEL - NSTEADY * SB


@functools.partial(
    pl.kernel,
    out_type=jax.ShapeDtypeStruct((NC, N, D), jnp.float32),
    mesh=_mesh,
    scratch_types=(
        [pltpu.VMEM((WIN,), jnp.int32)] * NR
        + [pltpu.VMEM((WIN,), jnp.int32)] * NI
        + [pltpu.VMEM((WIN, D), jnp.float32)] * NR
        + [pltpu.VMEM((TAIL,), jnp.int32)] * 2
        + [pltpu.VMEM_SHARED((N, D), jnp.float32)]
        + [pltpu.SemaphoreType.DMA] * (NR + NR + NI + 1)
    ),
)
def _sc_scatter(src_hbm, dst_hbm, g_hbm, zeros_hbm, out_hbm, *scr):
    srcb = scr[0:NR]
    dstb = scr[NR:NR + NI]
    rows = scr[NR + NI:2 * NR + NI]
    srct_v, dstt_v = scr[2 * NR + NI:2 * NR + NI + 2]
    acc_sh = scr[2 * NR + NI + 2]
    sem0 = 2 * NR + NI + 3
    sg = scr[sem0:sem0 + NR]
    ss = scr[sem0 + NR:sem0 + 2 * NR]
    si = scr[sem0 + 2 * NR:sem0 + 2 * NR + NI]
    semt = scr[sem0 + 2 * NR + NI]

    c = lax.axis_index("c")
    s = lax.axis_index("s")
    wkr = c * NS + s
    _rows_foreach_subcore(s, lambda off, nr: pltpu.sync_copy(
        zeros_hbm.at[pl.ds(off, nr)], acc_sh.at[pl.ds(off, nr)]))
    plsc.subcore_barrier()
    base = wkr * EPW

    def issue_idx(w, sw):
        pltpu.async_copy(src_hbm.at[pl.ds(base + w * WIN, WIN)],
                         srcb[sw % NR], si[sw % NI])
        pltpu.async_copy(dst_hbm.at[pl.ds(base + w * WIN, WIN)],
                         dstb[sw % NI], si[sw % NI])

    def wait_idx(sw):
        pltpu.make_async_copy(src_hbm.at[pl.ds(base, WIN)],
                              srcb[sw % NR], si[sw % NI]).wait()
        pltpu.make_async_copy(dst_hbm.at[pl.ds(base, WIN)],
                              dstb[sw % NI], si[sw % NI]).wait()

    def issue_gather(sw):
        pltpu.async_copy(g_hbm.at[srcb[sw % NR]], rows[sw % NR], sg[sw % NR])

    def wait_gather(sw):
        pltpu.make_async_copy(g_hbm.at[srcb[sw % NR]], rows[sw % NR],
                              sg[sw % NR]).wait()

    def issue_scatter(sw):
        pltpu.async_copy(rows[sw % NR], acc_sh.at[dstb[sw % NI]],
                         ss[sw % NR], add=True)

    def wait_scatter(sw):
        pltpu.make_async_copy(rows[sw % NR], acc_sh.at[dstb[sw % NI]],
                              ss[sw % NR]).wait()

    def body(w, sw, later_idx, later_gather, first):
        wait_gather(sw)
        issue_scatter(sw)
        if later_idx:
            issue_idx(w + NR, sw + NR)
        if not first:
            wait_scatter(sw - 1)
        if later_gather:
            wait_idx(sw + 2)
            issue_gather(sw + 2)

    for w in range(NR):
        issue_idx(w, w)
    for w in range(NR):
        wait_idx(w)
        issue_gather(w)

    for j in range(PEEL):
        body(j, j, later_idx=True, later_gather=(j >= 1), first=(j == 0))

    @pl.loop(0, NSTEADY)
    def _(k):
        w0 = PEEL + k * SB
        for j in range(SB):
            body(w0 + j, j, later_idx=True, later_gather=True, first=False)

    e0 = PEEL + NSTEADY * SB
    for j in range(e0, NWIN):
        body(j, j, later_idx=(j + NR < NWIN), later_gather=(j + 2 < NWIN),
             first=False)

    pltpu.async_copy(src_hbm.at[pl.ds(base + NWIN * WIN, TAIL)], srct_v, semt)
    pltpu.async_copy(dst_hbm.at[pl.ds(base + NWIN * WIN, TAIL)], dstt_v, semt)
    wait_scatter(NWIN - 1)
    pltpu.make_async_copy(src_hbm.at[pl.ds(base, TAIL)], srct_v, semt).wait()
    pltpu.make_async_copy(dst_hbm.at[pl.ds(base, TAIL)], dstt_v, semt).wait()
    tail_rows = rows[0].at[pl.ds(0, TAIL)]
    pltpu.async_copy(g_hbm.at[srct_v], tail_rows, semt)
    pltpu.make_async_copy(g_hbm.at[srct_v], tail_rows, semt).wait()
    pltpu.sync_copy(tail_rows, acc_sh.at[dstt_v], add=True)
    plsc.subcore_barrier()
    _rows_foreach_subcore(s, lambda off, nr: pltpu.sync_copy(
        acc_sh.at[pl.ds(off, nr)], out_hbm.at[c, pl.ds(off, nr)]))


BLK = 2000


def _tc_first_body(degp_ref, x_ref, w_ref, g_ref, dinv_ref):
    deg = degp_ref[0][:, 0:1] + degp_ref[1][:, 0:1] + 1.0
    dinv = lax.rsqrt(deg)
    t = jnp.dot(x_ref[...], w_ref[...], preferred_element_type=jnp.float32,
                precision=lax.Precision.HIGHEST)
    g_ref[...] = t * dinv
    dinv_ref[...] = dinv


def _tc_mid_body(accp_ref, g_ref, dinv_ref, b_ref, w_ref, gnext_ref):
    h = (accp_ref[0] + accp_ref[1] + g_ref[...]) * dinv_ref[...] + b_ref[...]
    h = jnp.maximum(h, 0.0)
    gnext_ref[...] = jnp.dot(h, w_ref[...], preferred_element_type=jnp.float32,
                             precision=lax.Precision.HIGHEST) * dinv_ref[...]


def _tc_last_body(accp_ref, g_ref, dinv_ref, b_ref, out_ref):
    out_ref[...] = (accp_ref[0] + accp_ref[1] + g_ref[...]) * dinv_ref[...] \
        + b_ref[...]


def _tc_first(degp, x, w):
    return pl.pallas_call(
        _tc_first_body,
        grid=(N // BLK,),
        in_specs=[
            pl.BlockSpec((2, BLK, D), lambda i: (0, i, 0)),
            pl.BlockSpec((BLK, D), lambda i: (i, 0)),
            pl.BlockSpec((D, D), lambda i: (0, 0)),
        ],
        out_specs=[
            pl.BlockSpec((BLK, D), lambda i: (i, 0)),
            pl.BlockSpec((BLK, 1), lambda i: (i, 0)),
        ],
        out_shape=[
            jax.ShapeDtypeStruct((N, D), jnp.float32),
            jax.ShapeDtypeStruct((N, 1), jnp.float32),
        ],
    )(degp, x, w)


def _tc_mid(accp, g, dinv, b, w):
    return pl.pallas_call(
        _tc_mid_body,
        grid=(N // BLK,),
        in_specs=[
            pl.BlockSpec((2, BLK, D), lambda i: (0, i, 0)),
            pl.BlockSpec((BLK, D), lambda i: (i, 0)),
            pl.BlockSpec((BLK, 1), lambda i: (i, 0)),
            pl.BlockSpec((1, D), lambda i: (0, 0)),
            pl.BlockSpec((D, D), lambda i: (0, 0)),
        ],
        out_specs=pl.BlockSpec((BLK, D), lambda i: (i, 0)),
        out_shape=jax.ShapeDtypeStruct((N, D), jnp.float32),
    )(accp, g, dinv, b, w)


def _tc_last(accp, g, dinv, b):
    return pl.pallas_call(
        _tc_last_body,
        grid=(N // BLK,),
        in_specs=[
            pl.BlockSpec((2, BLK, D), lambda i: (0, i, 0)),
            pl.BlockSpec((BLK, D), lambda i: (i, 0)),
            pl.BlockSpec((BLK, 1), lambda i: (i, 0)),
            pl.BlockSpec((1, D), lambda i: (0, 0)),
        ],
        out_specs=pl.BlockSpec((BLK, D), lambda i: (i, 0)),
        out_shape=jax.ShapeDtypeStruct((N, D), jnp.float32),
    )(accp, g, dinv, b)


def kernel(x, edge_index, W0, b0, W1, b1, W2, b2, W3, b3):
    src = edge_index[0]
    dst = edge_index[1]
    zerosD = jnp.zeros((N, D), jnp.float32)
    onesW = jnp.ones((WIN, D), jnp.float32)

    degp = _sc_degree(dst, zerosD, onesW)
    g, dinv = _tc_first(degp, x, W0)
    for w_next, b_prev in ((W1, b0), (W2, b1), (W3, b2)):
        accp = _sc_scatter(src, dst, g, zerosD)
        g = _tc_mid(accp, g, dinv, b_prev.reshape(1, D), w_next)
    accp = _sc_scatter(src, dst, g, zerosD)
    return _tc_last(accp, g, dinv, b3.reshape(1, D))


# speedup vs baseline: 24.8367x; 1.0157x over previous
import functools

import jax
import jax.numpy as jnp
from jax import lax
from jax.experimental import pallas as pl
from jax.experimental.pallas import tpu as pltpu
from jax.experimental.pallas import tpu_sc as plsc

N = 10000
E = 320000
D = 128

NC = 2
NS = 16
NW = NC * NS
EPW = E // NW
WIN = 128
NWIN = EPW // WIN
TAIL = EPW - NWIN * WIN
RPS = 624
RPS_LAST = N - 15 * RPS

_mesh = plsc.VectorSubcoreMesh(core_axis_name="c", subcore_axis_name="s")


def _rows_foreach_subcore(s, fn):
    off = pl.multiple_of(s * RPS, 8)

    @pl.when(s < NS - 1)
    def _():
        fn(off, RPS)

    @pl.when(s == NS - 1)
    def _():
        fn(off, RPS_LAST)


DNI = 8
DSB = 8


@functools.partial(
    pl.kernel,
    out_type=jax.ShapeDtypeStruct((NC, N, D), jnp.float32),
    mesh=_mesh,
    scratch_types=(
        [pltpu.VMEM((WIN,), jnp.int32)] * DNI
        + [pltpu.VMEM((TAIL,), jnp.int32)]
        + [pltpu.VMEM((WIN, D), jnp.float32)]
        + [pltpu.VMEM((TAIL, D), jnp.float32)]
        + [pltpu.VMEM_SHARED((N, D), jnp.float32)]
        + [pltpu.SemaphoreType.DMA] * (2 * DNI + 1)
    ),
)
def _sc_degree(dst_hbm, zeros_hbm, ones_hbm, out_hbm, *scr):
    dstb = scr[0:DNI]
    idxt_v = scr[DNI]
    ones_v = scr[DNI + 1]
    onest_v = scr[DNI + 2]
    acc_sh = scr[DNI + 3]
    si = scr[DNI + 4:2 * DNI + 4]
    ss = scr[2 * DNI + 4:3 * DNI + 4]
    semt = scr[3 * DNI + 4]

    c = lax.axis_index("c")
    s = lax.axis_index("s")
    wkr = c * NS + s
    base = wkr * EPW

    def issue_idx(w, sw):
        pltpu.async_copy(dst_hbm.at[pl.ds(base + w * WIN, WIN)],
                         dstb[sw % DNI], si[sw % DNI])

    def wait_idx(sw):
        pltpu.make_async_copy(dst_hbm.at[pl.ds(base, WIN)],
                              dstb[sw % DNI], si[sw % DNI]).wait()

    def issue_scatter(sw):
        pltpu.async_copy(ones_v, acc_sh.at[dstb[sw % DNI]], ss[sw % DNI],
                         add=True)

    def wait_scatter(sw):
        pltpu.make_async_copy(ones_v, acc_sh.at[dstb[sw % DNI]],
                              ss[sw % DNI]).wait()

    def body(w, sw, later_idx, first):
        wait_idx(sw)
        issue_scatter(sw)
        if later_idx:
            if not first:
                wait_scatter(sw + 4)
            issue_idx(w + 4, sw + 4)

    for w in range(4):
        issue_idx(w, w)
    pltpu.sync_copy(ones_hbm, ones_v)
    pltpu.sync_copy(ones_hbm.at[pl.ds(0, TAIL)], onest_v)
    _rows_foreach_subcore(s, lambda off, nr: pltpu.sync_copy(
        zeros_hbm.at[pl.ds(off, nr)], acc_sh.at[pl.ds(off, nr)]))
    plsc.subcore_barrier()
    for j in range(DSB):
        body(j, j, later_idx=True, first=(j < 4))

    @pl.loop(0, (NWIN - DSB - 6) // DSB)
    def _(k):
        w0 = DSB + k * DSB
        for j in range(DSB):
            body(w0 + j, j, later_idx=True, first=False)

    e0 = DSB + ((NWIN - DSB - 6) // DSB) * DSB
    for j in range(e0, NWIN):
        body(j, j, later_idx=(j + 4 < NWIN), first=False)

    pltpu.async_copy(dst_hbm.at[pl.ds(base + NWIN * WIN, TAIL)], idxt_v, semt)
    pltpu.make_async_copy(dst_hbm.at[pl.ds(base, TAIL)], idxt_v, semt).wait()
    pltpu.sync_copy(onest_v, acc_sh.at[idxt_v], add=True)
    for w in range(NWIN - 8, NWIN):
        wait_scatter(w % DSB)
    plsc.subcore_barrier()
    _rows_foreach_subcore(s, lambda off, nr: pltpu.sync_copy(
        acc_sh.at[pl.ds(off, nr)], out_hbm.at[c, pl.ds(off, nr)]))


NR = 3
NI = 8
SB = 24
PEEL = SB
NSTEADY = (NWIN - PEEL - 6) // SB
EPI = NWIN - PEEL - NSTEADY * SB


@functools.partial(
    pl.kernel,
    out_type=jax.ShapeDtypeStruct((NC, N, D), jnp.float32),
    mesh=_mesh,
    scratch_types=(
        [pltpu.VMEM((WIN,), jnp.int32)] * NR
        + [pltpu.VMEM((WIN,), jnp.int32)] * NI
        + [pltpu.VMEM((WIN, D), jnp.float32)] * NR
        + [pltpu.VMEM((TAIL,), jnp.int32)] * 2
        + [pltpu.VMEM_SHARED((N, D), jnp.float32)]
        + [pltpu.SemaphoreType.DMA] * (NR + NR + NI + 1)
    ),
)
def _sc_scatter(src_hbm, dst_hbm, g_hbm, zeros_hbm, out_hbm, *scr):
    srcb = scr[0:NR]
    dstb = scr[NR:NR + NI]
    rows = scr[NR + NI:2 * NR + NI]
    srct_v, dstt_v = scr[2 * NR + NI:2 * NR + NI + 2]
    acc_sh = scr[2 * NR + NI + 2]
    sem0 = 2 * NR + NI + 3
    sg = scr[sem0:sem0 + NR]
    ss = scr[sem0 + NR:sem0 + 2 * NR]
    si = scr[sem0 + 2 * NR:sem0 + 2 * NR + NI]
    semt = scr[sem0 + 2 * NR + NI]

    c = lax.axis_index("c")
    s = lax.axis_index("s")
    wkr = c * NS + s
    base = wkr * EPW

    def issue_idx(w, sw):
        pltpu.async_copy(src_hbm.at[pl.ds(base + w * WIN, WIN)],
                         srcb[sw % NR], si[sw % NI])
        pltpu.async_copy(dst_hbm.at[pl.ds(base + w * WIN, WIN)],
                         dstb[sw % NI], si[sw % NI])

    def wait_idx(sw):
        pltpu.make_async_copy(src_hbm.at[pl.ds(base, WIN)],
                              srcb[sw % NR], si[sw % NI]).wait()
        pltpu.make_async_copy(dst_hbm.at[pl.ds(base, WIN)],
                              dstb[sw % NI], si[sw % NI]).wait()

    def issue_gather(sw):
        pltpu.async_copy(g_hbm.at[srcb[sw % NR]], rows[sw % NR], sg[sw % NR])

    def wait_gather(sw):
        pltpu.make_async_copy(g_hbm.at[srcb[sw % NR]], rows[sw % NR],
                              sg[sw % NR]).wait()

    def issue_scatter(sw):
        pltpu.async_copy(rows[sw % NR], acc_sh.at[dstb[sw % NI]],
                         ss[sw % NR], add=True)

    def wait_scatter(sw):
        pltpu.make_async_copy(rows[sw % NR], acc_sh.at[dstb[sw % NI]],
                              ss[sw % NR]).wait()

    def body(w, sw, later_idx, later_gather, first):
        wait_gather(sw)
        issue_scatter(sw)
        if later_idx:
            issue_idx(w + NR, sw + NR)
        if not first:
            wait_scatter(sw - 1)
        if later_gather:
            wait_idx(sw + 2)
            issue_gather(sw + 2)

    for w in range(NR):
        issue_idx(w, w)
    _rows_foreach_subcore(s, lambda off, nr: pltpu.sync_copy(
        zeros_hbm.at[pl.ds(off, nr)], acc_sh.at[pl.ds(off, nr)]))
    for w in range(NR):
        wait_idx(w)
        issue_gather(w)
    plsc.subcore_barrier()

    for j in range(PEEL):
        body(j, j, later_idx=True, later_gather=(j >= 1), first=(j == 0))

    @pl.loop(0, NSTEADY)
    def _(k):
        w0 = PEEL + k * SB
        for j in range(SB):
            body(w0 + j, j, later_idx=True, later_gather=True, first=False)

    e0 = PEEL + NSTEADY * SB
    for j in range(e0, NWIN):
        body(j, j, later_idx=(j + NR < NWIN), later_gather=(j + 2 < NWIN),
             first=False)

    pltpu.async_copy(src_hbm.at[pl.ds(base + NWIN * WIN, TAIL)], srct_v, semt)
    pltpu.async_copy(dst_hbm.at[pl.ds(base + NWIN * WIN, TAIL)], dstt_v, semt)
    wait_scatter(NWIN - 1)
    pltpu.make_async_copy(src_hbm.at[pl.ds(base, TAIL)], srct_v, semt).wait()
    pltpu.make_async_copy(dst_hbm.at[pl.ds(base, TAIL)], dstt_v, semt).wait()
    tail_rows = rows[0].at[pl.ds(0, TAIL)]
    pltpu.async_copy(g_hbm.at[srct_v], tail_rows, semt)
    pltpu.make_async_copy(g_hbm.at[srct_v], tail_rows, semt).wait()
    pltpu.sync_copy(tail_rows, acc_sh.at[dstt_v], add=True)
    plsc.subcore_barrier()
    _rows_foreach_subcore(s, lambda off, nr: pltpu.sync_copy(
        acc_sh.at[pl.ds(off, nr)], out_hbm.at[c, pl.ds(off, nr)]))


BLK = 2000


def _tc_first_body(degp_ref, x_ref, w_ref, g_ref, dinv_ref):
    deg = degp_ref[0][:, 0:1] + degp_ref[1][:, 0:1] + 1.0
    dinv = lax.rsqrt(deg)
    t = jnp.dot(x_ref[...], w_ref[...], preferred_element_type=jnp.float32,
                precision=lax.Precision.HIGHEST)
    g_ref[...] = t * dinv
    dinv_ref[...] = dinv


def _tc_mid_body(accp_ref, g_ref, dinv_ref, b_ref, w_ref, gnext_ref):
    h = (accp_ref[0] + accp_ref[1] + g_ref[...]) * dinv_ref[...] + b_ref[...]
    h = jnp.maximum(h, 0.0)
    gnext_ref[...] = jnp.dot(h, w_ref[...], preferred_element_type=jnp.float32,
                             precision=lax.Precision.HIGHEST) * dinv_ref[...]


def _tc_last_body(accp_ref, g_ref, dinv_ref, b_ref, out_ref):
    out_ref[...] = (accp_ref[0] + accp_ref[1] + g_ref[...]) * dinv_ref[...] \
        + b_ref[...]


def _tc_first(degp, x, w):
    return pl.pallas_call(
        _tc_first_body,
        grid=(N // BLK,),
        in_specs=[
            pl.BlockSpec((2, BLK, D), lambda i: (0, i, 0)),
            pl.BlockSpec((BLK, D), lambda i: (i, 0)),
            pl.BlockSpec((D, D), lambda i: (0, 0)),
        ],
        out_specs=[
            pl.BlockSpec((BLK, D), lambda i: (i, 0)),
            pl.BlockSpec((BLK, 1), lambda i: (i, 0)),
        ],
        out_shape=[
            jax.ShapeDtypeStruct((N, D), jnp.float32),
            jax.ShapeDtypeStruct((N, 1), jnp.float32),
        ],
    )(degp, x, w)


def _tc_mid(accp, g, dinv, b, w):
    return pl.pallas_call(
        _tc_mid_body,
        grid=(N // BLK,),
        in_specs=[
            pl.BlockSpec((2, BLK, D), lambda i: (0, i, 0)),
            pl.BlockSpec((BLK, D), lambda i: (i, 0)),
            pl.BlockSpec((BLK, 1), lambda i: (i, 0)),
            pl.BlockSpec((1, D), lambda i: (0, 0)),
            pl.BlockSpec((D, D), lambda i: (0, 0)),
        ],
        out_specs=pl.BlockSpec((BLK, D), lambda i: (i, 0)),
        out_shape=jax.ShapeDtypeStruct((N, D), jnp.float32),
    )(accp, g, dinv, b, w)


def _tc_last(accp, g, dinv, b):
    return pl.pallas_call(
        _tc_last_body,
        grid=(N // BLK,),
        in_specs=[
            pl.BlockSpec((2, BLK, D), lambda i: (0, i, 0)),
            pl.BlockSpec((BLK, D), lambda i: (i, 0)),
            pl.BlockSpec((BLK, 1), lambda i: (i, 0)),
            pl.BlockSpec((1, D), lambda i: (0, 0)),
        ],
        out_specs=pl.BlockSpec((BLK, D), lambda i: (i, 0)),
        out_shape=jax.ShapeDtypeStruct((N, D), jnp.float32),
    )(accp, g, dinv, b)


def kernel(x, edge_index, W0, b0, W1, b1, W2, b2, W3, b3):
    src = edge_index[0]
    dst = edge_index[1]
    zerosD = jnp.zeros((N, D), jnp.float32)
    onesW = jnp.ones((WIN, D), jnp.float32)

    degp = _sc_degree(dst, zerosD, onesW)
    g, dinv = _tc_first(degp, x, W0)
    for w_next, b_prev in ((W1, b0), (W2, b1), (W3, b2)):
        accp = _sc_scatter(src, dst, g, zerosD)
        g = _tc_mid(accp, g, dinv, b_prev.reshape(1, D), w_next)
    accp = _sc_scatter(src, dst, g, zerosD)
    return _tc_last(accp, g, dinv, b3.reshape(1, D))


# speedup vs baseline: 24.8691x; 1.0013x over previous
import functools

import jax
import jax.numpy as jnp
from jax import lax
from jax.experimental import pallas as pl
from jax.experimental.pallas import tpu as pltpu
from jax.experimental.pallas import tpu_sc as plsc

N = 10000
E = 320000
D = 128

NC = 2
NS = 16
NW = NC * NS
EPW = E // NW
WIN = 128
NWIN = EPW // WIN
TAIL = EPW - NWIN * WIN
RPS = 624
RPS_LAST = N - 15 * RPS

_mesh = plsc.VectorSubcoreMesh(core_axis_name="c", subcore_axis_name="s")


def _rows_foreach_subcore(s, fn):
    off = pl.multiple_of(s * RPS, 8)

    @pl.when(s < NS - 1)
    def _():
        fn(off, RPS)

    @pl.when(s == NS - 1)
    def _():
        fn(off, RPS_LAST)


DNI = 8
DSB = 8


@functools.partial(
    pl.kernel,
    out_type=jax.ShapeDtypeStruct((NC, N, D), jnp.float32),
    mesh=_mesh,
    scratch_types=(
        [pltpu.VMEM((WIN,), jnp.int32)] * DNI
        + [pltpu.VMEM((TAIL,), jnp.int32)]
        + [pltpu.VMEM((WIN, D), jnp.float32)]
        + [pltpu.VMEM((TAIL, D), jnp.float32)]
        + [pltpu.VMEM_SHARED((N, D), jnp.float32)]
        + [pltpu.SemaphoreType.DMA] * (2 * DNI + 1)
    ),
)
def _sc_degree(dst_hbm, zeros_hbm, ones_hbm, out_hbm, *scr):
    dstb = scr[0:DNI]
    idxt_v = scr[DNI]
    ones_v = scr[DNI + 1]
    onest_v = scr[DNI + 2]
    acc_sh = scr[DNI + 3]
    si = scr[DNI + 4:2 * DNI + 4]
    ss = scr[2 * DNI + 4:3 * DNI + 4]
    semt = scr[3 * DNI + 4]

    c = lax.axis_index("c")
    s = lax.axis_index("s")
    wkr = c * NS + s
    base = wkr * EPW

    def issue_idx(w, sw):
        pltpu.async_copy(dst_hbm.at[pl.ds(base + w * WIN, WIN)],
                         dstb[sw % DNI], si[sw % DNI])

    def wait_idx(sw):
        pltpu.make_async_copy(dst_hbm.at[pl.ds(base, WIN)],
                              dstb[sw % DNI], si[sw % DNI]).wait()

    def issue_scatter(sw):
        pltpu.async_copy(ones_v, acc_sh.at[dstb[sw % DNI]], ss[sw % DNI],
                         add=True)

    def wait_scatter(sw):
        pltpu.make_async_copy(ones_v, acc_sh.at[dstb[sw % DNI]],
                              ss[sw % DNI]).wait()

    def body(w, sw, later_idx, first):
        wait_idx(sw)
        issue_scatter(sw)
        if later_idx:
            if not first:
                wait_scatter(sw + 4)
            issue_idx(w + 4, sw + 4)

    for w in range(4):
        issue_idx(w, w)
    pltpu.sync_copy(ones_hbm, ones_v)
    pltpu.sync_copy(ones_hbm.at[pl.ds(0, TAIL)], onest_v)
    _rows_foreach_subcore(s, lambda off, nr: pltpu.sync_copy(
        zeros_hbm.at[pl.ds(off, nr)], acc_sh.at[pl.ds(off, nr)]))
    plsc.subcore_barrier()
    for j in range(DSB):
        body(j, j, later_idx=True, first=(j < 4))

    @pl.loop(0, (NWIN - DSB - 6) // DSB)
    def _(k):
        w0 = DSB + k * DSB
        for j in range(DSB):
            body(w0 + j, j, later_idx=True, first=False)

    e0 = DSB + ((NWIN - DSB - 6) // DSB) * DSB
    for j in range(e0, NWIN):
        body(j, j, later_idx=(j + 4 < NWIN), first=False)

    pltpu.async_copy(dst_hbm.at[pl.ds(base + NWIN * WIN, TAIL)], idxt_v, semt)
    pltpu.make_async_copy(dst_hbm.at[pl.ds(base, TAIL)], idxt_v, semt).wait()
    pltpu.sync_copy(onest_v, acc_sh.at[idxt_v], add=True)
    for w in range(NWIN - 8, NWIN):
        wait_scatter(w % DSB)
    plsc.subcore_barrier()
    _rows_foreach_subcore(s, lambda off, nr: pltpu.sync_copy(
        acc_sh.at[pl.ds(off, nr)], out_hbm.at[c, pl.ds(off, nr)]))


NR = 3
NI = 8
SB = 24
PEEL = SB
NSTEADY = (NWIN - PEEL - 6) // SB
EPI = NWIN - PEEL - NSTEADY * SB


@functools.partial(
    pl.kernel,
    out_type=jax.ShapeDtypeStruct((NC, N, D), jnp.float32),
    mesh=_mesh,
    scratch_types=(
        [pltpu.VMEM((WIN,), jnp.int32)] * NR
        + [pltpu.VMEM((WIN,), jnp.int32)] * NI
        + [pltpu.VMEM((WIN, D), jnp.float32)] * NR
        + [pltpu.VMEM((TAIL,), jnp.int32)] * 2
        + [pltpu.VMEM_SHARED((N, D), jnp.float32)]
        + [pltpu.SemaphoreType.DMA] * (NR + NR + NI + 1)
    ),
)
def _sc_scatter(src_hbm, dst_hbm, g_hbm, zeros_hbm, out_hbm, *scr):
    srcb = scr[0:NR]
    dstb = scr[NR:NR + NI]
    rows = scr[NR + NI:2 * NR + NI]
    srct_v, dstt_v = scr[2 * NR + NI:2 * NR + NI + 2]
    acc_sh = scr[2 * NR + NI + 2]
    sem0 = 2 * NR + NI + 3
    sg = scr[sem0:sem0 + NR]
    ss = scr[sem0 + NR:sem0 + 2 * NR]
    si = scr[sem0 + 2 * NR:sem0 + 2 * NR + NI]
    semt = scr[sem0 + 2 * NR + NI]

    c = lax.axis_index("c")
    s = lax.axis_index("s")
    wkr = c * NS + s
    base = wkr * EPW

    def issue_idx(w, sw):
        pltpu.async_copy(src_hbm.at[pl.ds(base + w * WIN, WIN)],
                         srcb[sw % NR], si[sw % NI])
        pltpu.async_copy(dst_hbm.at[pl.ds(base + w * WIN, WIN)],
                         dstb[sw % NI], si[sw % NI])

    def wait_idx(sw):
        pltpu.make_async_copy(src_hbm.at[pl.ds(base, WIN)],
                              srcb[sw % NR], si[sw % NI]).wait()
        pltpu.make_async_copy(dst_hbm.at[pl.ds(base, WIN)],
                              dstb[sw % NI], si[sw % NI]).wait()

    def issue_gather(sw):
        pltpu.async_copy(g_hbm.at[srcb[sw % NR]], rows[sw % NR], sg[sw % NR])

    def wait_gather(sw):
        pltpu.make_async_copy(g_hbm.at[srcb[sw % NR]], rows[sw % NR],
                              sg[sw % NR]).wait()

    def issue_scatter(sw):
        pltpu.async_copy(rows[sw % NR], acc_sh.at[dstb[sw % NI]],
                         ss[sw % NR], add=True)

    def wait_scatter(sw):
        pltpu.make_async_copy(rows[sw % NR], acc_sh.at[dstb[sw % NI]],
                              ss[sw % NR]).wait()

    def body(w, sw, later_idx, later_gather, first):
        wait_gather(sw)
        issue_scatter(sw)
        if later_idx:
            issue_idx(w + NR, sw + NR)
        if not first:
            wait_scatter(sw - 1)
        if later_gather:
            wait_idx(sw + 2)
            issue_gather(sw + 2)

    for w in range(NR):
        issue_idx(w, w)
    _rows_foreach_subcore(s, lambda off, nr: pltpu.sync_copy(
        zeros_hbm.at[pl.ds(off, nr)], acc_sh.at[pl.ds(off, nr)]))
    for w in range(NR):
        wait_idx(w)
        issue_gather(w)
    plsc.subcore_barrier()

    for j in range(PEEL):
        body(j, j, later_idx=True, later_gather=(j >= 1), first=(j == 0))

    @pl.loop(0, NSTEADY)
    def _(k):
        w0 = PEEL + k * SB
        for j in range(SB):
            body(w0 + j, j, later_idx=True, later_gather=True, first=False)

    e0 = PEEL + NSTEADY * SB
    for j in range(e0, NWIN):
        body(j, j, later_idx=(j + NR < NWIN), later_gather=(j + 2 < NWIN),
             first=False)

    pltpu.async_copy(src_hbm.at[pl.ds(base + NWIN * WIN, TAIL)], srct_v, semt)
    pltpu.async_copy(dst_hbm.at[pl.ds(base + NWIN * WIN, TAIL)], dstt_v, semt)
    wait_scatter(NWIN - 1)
    pltpu.make_async_copy(src_hbm.at[pl.ds(base, TAIL)], srct_v, semt).wait()
    pltpu.make_async_copy(dst_hbm.at[pl.ds(base, TAIL)], dstt_v, semt).wait()
    tail_rows = rows[0].at[pl.ds(0, TAIL)]
    pltpu.async_copy(g_hbm.at[srct_v], tail_rows, semt)
    pltpu.make_async_copy(g_hbm.at[srct_v], tail_rows, semt).wait()
    pltpu.sync_copy(tail_rows, acc_sh.at[dstt_v], add=True)
    plsc.subcore_barrier()
    _rows_foreach_subcore(s, lambda off, nr: pltpu.sync_copy(
        acc_sh.at[pl.ds(off, nr)], out_hbm.at[c, pl.ds(off, nr)]))


BLK = 2000


def _tc_matmul_body(x_ref, w_ref, t_ref):
    t_ref[...] = jnp.dot(x_ref[...], w_ref[...],
                         preferred_element_type=jnp.float32,
                         precision=lax.Precision.HIGHEST)


def _tc_first_body(degp_ref, t_ref, g_ref, dinv_ref):
    deg = degp_ref[0][:, 0:1] + degp_ref[1][:, 0:1] + 1.0
    dinv = lax.rsqrt(deg)
    g_ref[...] = t_ref[...] * dinv
    dinv_ref[...] = dinv


def _tc_mid_body(accp_ref, g_ref, dinv_ref, b_ref, w_ref, gnext_ref):
    h = (accp_ref[0] + accp_ref[1] + g_ref[...]) * dinv_ref[...] + b_ref[...]
    h = jnp.maximum(h, 0.0)
    gnext_ref[...] = jnp.dot(h, w_ref[...], preferred_element_type=jnp.float32,
                             precision=lax.Precision.HIGHEST) * dinv_ref[...]


def _tc_last_body(accp_ref, g_ref, dinv_ref, b_ref, out_ref):
    out_ref[...] = (accp_ref[0] + accp_ref[1] + g_ref[...]) * dinv_ref[...] \
        + b_ref[...]


def _tc_matmul(x, w):
    return pl.pallas_call(
        _tc_matmul_body,
        grid=(N // BLK,),
        in_specs=[
            pl.BlockSpec((BLK, D), lambda i: (i, 0)),
            pl.BlockSpec((D, D), lambda i: (0, 0)),
        ],
        out_specs=pl.BlockSpec((BLK, D), lambda i: (i, 0)),
        out_shape=jax.ShapeDtypeStruct((N, D), jnp.float32),
    )(x, w)


def _tc_first(degp, t):
    return pl.pallas_call(
        _tc_first_body,
        grid=(N // BLK,),
        in_specs=[
            pl.BlockSpec((2, BLK, D), lambda i: (0, i, 0)),
            pl.BlockSpec((BLK, D), lambda i: (i, 0)),
        ],
        out_specs=[
            pl.BlockSpec((BLK, D), lambda i: (i, 0)),
            pl.BlockSpec((BLK, 1), lambda i: (i, 0)),
        ],
        out_shape=[
            jax.ShapeDtypeStruct((N, D), jnp.float32),
            jax.ShapeDtypeStruct((N, 1), jnp.float32),
        ],
    )(degp, t)


def _tc_mid(accp, g, dinv, b, w):
    return pl.pallas_call(
        _tc_mid_body,
        grid=(N // BLK,),
        in_specs=[
            pl.BlockSpec((2, BLK, D), lambda i: (0, i, 0)),
            pl.BlockSpec((BLK, D), lambda i: (i, 0)),
            pl.BlockSpec((BLK, 1), lambda i: (i, 0)),
            pl.BlockSpec((1, D), lambda i: (0, 0)),
            pl.BlockSpec((D, D), lambda i: (0, 0)),
        ],
        out_specs=pl.BlockSpec((BLK, D), lambda i: (i, 0)),
        out_shape=jax.ShapeDtypeStruct((N, D), jnp.float32),
    )(accp, g, dinv, b, w)


def _tc_last(accp, g, dinv, b):
    return pl.pallas_call(
        _tc_last_body,
        grid=(N // BLK,),
        in_specs=[
            pl.BlockSpec((2, BLK, D), lambda i: (0, i, 0)),
            pl.BlockSpec((BLK, D), lambda i: (i, 0)),
            pl.BlockSpec((BLK, 1), lambda i: (i, 0)),
            pl.BlockSpec((1, D), lambda i: (0, 0)),
        ],
        out_specs=pl.BlockSpec((BLK, D), lambda i: (i, 0)),
        out_shape=jax.ShapeDtypeStruct((N, D), jnp.float32),
    )(accp, g, dinv, b)


def kernel(x, edge_index, W0, b0, W1, b1, W2, b2, W3, b3):
    src = edge_index[0]
    dst = edge_index[1]
    zerosD = jnp.zeros((N, D), jnp.float32)
    onesW = jnp.ones((WIN, D), jnp.float32)

    degp = _sc_degree(dst, zerosD, onesW)
    t0 = _tc_matmul(x, W0)
    g, dinv = _tc_first(degp, t0)
    for w_next, b_prev in ((W1, b0), (W2, b1), (W3, b2)):
        accp = _sc_scatter(src, dst, g, zerosD)
        g = _tc_mid(accp, g, dinv, b_prev.reshape(1, D), w_next)
    accp = _sc_scatter(src, dst, g, zerosD)
    return _tc_last(accp, g, dinv, b3.reshape(1, D))
